# baseline (speedup 1.0000x reference)
import functools

import jax
import jax.numpy as jnp
from jax import lax
from jax.experimental import pallas as pl
from jax.experimental.pallas import tpu as pltpu
from jax.experimental.pallas import tpu_sc as plsc

NEG_SLOPE = 0.2
CHUNK = 80
NC, NS, L = 2, 16, 16


def _proj_body(x_ref, w_ref, ad_ref, as_ref, h_ref, sd_ref, ss_ref, c_ref,
               msd_ref, mss_ref):
    i = pl.program_id(0)
    h = jnp.dot(x_ref[...], w_ref[...], preferred_element_type=jnp.float32)
    h_ref[...] = h
    sd = jnp.dot(h, ad_ref[...], preferred_element_type=jnp.float32)
    ss = jnp.dot(h, as_ref[...], preferred_element_type=jnp.float32)
    sd_ref[...] = sd
    ss_ref[...] = ss
    msd = jnp.broadcast_to(jnp.max(sd), (1, 128))
    mss = jnp.broadcast_to(jnp.max(ss), (1, 128))

    @pl.when(i == 0)
    def _():
        msd_ref[...] = msd
        mss_ref[...] = mss

    @pl.when(i > 0)
    def _():
        msd_ref[...] = jnp.maximum(msd_ref[...], msd)
        mss_ref[...] = jnp.maximum(mss_ref[...], mss)

    z = msd_ref[...] + mss_ref[...]
    c_ref[...] = jnp.maximum(z, jnp.float32(NEG_SLOPE) * z)


def _project(x, W, a_dst, a_src, block_rows=1000):
    n, d = x.shape
    dout = W.shape[1]
    grid = n // block_rows
    return pl.pallas_call(
        _proj_body,
        grid=(grid,),
        in_specs=[
            pl.BlockSpec((block_rows, d), lambda i: (i, 0)),
            pl.BlockSpec((d, dout), lambda i: (0, 0)),
            pl.BlockSpec((dout, 1), lambda i: (0, 0)),
            pl.BlockSpec((dout, 1), lambda i: (0, 0)),
        ],
        out_specs=[
            pl.BlockSpec((block_rows, dout), lambda i: (i, 0)),
            pl.BlockSpec((block_rows, 1), lambda i: (i, 0)),
            pl.BlockSpec((block_rows, 1), lambda i: (i, 0)),
            pl.BlockSpec((1, 128), lambda i: (0, 0)),
        ],
        out_shape=[
            jax.ShapeDtypeStruct((n, dout), jnp.float32),
            jax.ShapeDtypeStruct((n, 1), jnp.float32),
            jax.ShapeDtypeStruct((n, 1), jnp.float32),
            jax.ShapeDtypeStruct((1, 128), jnp.float32),
        ],
        scratch_shapes=[
            pltpu.VMEM((1, 128), jnp.float32),
            pltpu.VMEM((1, 128), jnp.float32),
        ],
    )(x, W, a_dst, a_src)


def _make_sc_kernel(n, e, d):
    epw = e // (NC * NS)
    nchunks = epw // CHUNK
    assert nchunks % 2 == 1, "pipeline assumes odd chunk count"
    pairs = (nchunks - 1) // 2
    groups = CHUNK // L
    drows = -(-(-(-n // 128)) // 8) * 8
    assert drows == CHUNK
    rows_per_tile = (-(-(n // NS) // 8) + 1) * 8
    assert (n // NS) * (NS - 1) // 8 * 8 + rows_per_tile >= n
    nzfull = rows_per_tile // CHUNK
    nzrem = rows_per_tile - nzfull * CHUNK

    mesh = plsc.VectorSubcoreMesh(core_axis_name="c", subcore_axis_name="s")

    @functools.partial(
        pl.kernel,
        out_type=jax.ShapeDtypeStruct((NC, n + drows, d), jnp.float32),
        mesh=mesh,
        scratch_types=[
            pltpu.VMEM((drows, 128), jnp.float32),
            pltpu.VMEM((CHUNK, d), jnp.float32),
            pltpu.VMEM((CHUNK, d), jnp.float32),
            pltpu.VMEM((2 * CHUNK,), jnp.int32),
            pltpu.VMEM((CHUNK,), jnp.int32),
            pltpu.VMEM((CHUNK,), jnp.int32),
            pltpu.VMEM((2 * CHUNK,), jnp.float32),
            pltpu.VMEM((2 * CHUNK,), jnp.float32),
            pltpu.VMEM((CHUNK,), jnp.float32),
            pltpu.VMEM((L,), jnp.float32),
            pltpu.VMEM_SHARED((n + drows, d), jnp.float32),
            pltpu.SemaphoreType.DMA,
            pltpu.SemaphoreType.DMA,
            pltpu.SemaphoreType.DMA,
            pltpu.SemaphoreType.DMA,
            pltpu.SemaphoreType.DMA,
            pltpu.SemaphoreType.DMA,
            pltpu.SemaphoreType.DMA,
            pltpu.SemaphoreType.DMA,
            pltpu.SemaphoreType.DMA,
            pltpu.SemaphoreType.DMA,
            pltpu.SemaphoreType.DMA,
            pltpu.SemaphoreType.DMA,
        ],
        compiler_params=pltpu.CompilerParams(needs_layout_passes=False),
    )
    def sc_gat(h_hbm, src_hbm, dst_hbm, sd_hbm, ss_hbm, c_hbm, out_hbm,
               den_v, h0_v, h1_v, srcb_v, dst0_v, dst1_v, sdb_v, ssb_v,
               p_v, cb_v, acc_sh,
               ssem0, ssem1, dsem0, dsem1, gsem0, gsem1,
               sdsem0, sdsem1, sssem0, sssem1, scsem0, scsem1):
        c = lax.axis_index("c")
        s = lax.axis_index("s")
        wid = s * NC + c
        ebase = wid * epw

        zeros16 = jnp.zeros((L,), jnp.float32)
        zeros16i = jnp.zeros((L,), jnp.int32)

        def zero_row(r, _):
            for j in range(d // L):
                h0_v[r, pl.ds(j * L, L)] = zeros16
            return 0

        lax.fori_loop(0, CHUNK, zero_row, 0)

        def zero_den(r, _):
            for j in range(128 // L):
                den_v[r, pl.ds(j * L, L)] = zeros16
            return 0

        lax.fori_loop(0, drows, zero_den, 0)

        rbase = (s * (n // NS)) // 8 * 8
        for k in range(nzfull):
            pltpu.sync_copy(h0_v, acc_sh.at[pl.ds(rbase + k * CHUNK, CHUNK)])
        if nzrem:
            pltpu.sync_copy(h0_v.at[pl.ds(0, nzrem)],
                            acc_sh.at[pl.ds(rbase + nzfull * CHUNK, nzrem)])

        @pl.when(s == 0)
        def _():
            pltpu.sync_copy(h0_v, acc_sh.at[pl.ds(n, drows)])

        pltpu.sync_copy(c_hbm.at[0, pl.ds(0, L)], cb_v)
        cshift = cb_v[pl.ds(0, L)]

        plsc.subcore_barrier()

        hbufs = (h0_v, h1_v)
        dstbufs = (dst0_v, dst1_v)
        ssems = (ssem0, ssem1)
        dsems = (dsem0, dsem1)
        gsems = (gsem0, gsem1)
        sdsems = (sdsem0, sdsem1)
        sssems = (sssem0, sssem1)
        scsems = (scsem0, scsem1)
        last = nchunks - 1

        def srcslice(b):
            return srcb_v.at[pl.ds(b * CHUNK, CHUNK)]

        def sdslice(b):
            return sdb_v.at[pl.ds(b * CHUNK, CHUNK)]

        def ssslice(b):
            return ssb_v.at[pl.ds(b * CHUNK, CHUNK)]

        def fire_src(ci, b):
            off = ebase + jnp.minimum(ci, last) * CHUNK
            pltpu.async_copy(src_hbm.at[pl.ds(off, CHUNK)], srcslice(b),
                             ssems[b])

        def wait_src(b):
            pltpu.make_async_copy(src_hbm.at[pl.ds(0, CHUNK)], srcslice(b),
                                  ssems[b]).wait()

        def fire_dst(ci, b):
            off = ebase + jnp.minimum(ci, last) * CHUNK
            pltpu.async_copy(dst_hbm.at[pl.ds(off, CHUNK)], dstbufs[b],
                             dsems[b])

        def wait_dst(b):
            pltpu.make_async_copy(dst_hbm.at[pl.ds(0, CHUNK)], dstbufs[b],
                                  dsems[b]).wait()

        def fire_gather(b):
            pltpu.async_copy(h_hbm.at[srcslice(b)], hbufs[b], gsems[b])

        def wait_gather(b):
            pltpu.make_async_copy(h_hbm.at[srcslice(b)], hbufs[b],
                                  gsems[b]).wait()

        def fire_sd(b):
            pltpu.async_copy(sd_hbm.at[dstbufs[b]], sdslice(b), sdsems[b])

        def wait_sd(b):
            pltpu.make_async_copy(sd_hbm.at[dstbufs[b]], sdslice(b),
                                  sdsems[b]).wait()

        def fire_ss(b):
            pltpu.async_copy(ss_hbm.at[srcslice(b)], ssslice(b), sssems[b])

        def wait_ss(b):
            pltpu.make_async_copy(ss_hbm.at[srcslice(b)], ssslice(b),
                                  sssems[b]).wait()

        def score(b):
            for g in range(groups):
                sl = pl.ds(b * CHUNK + g * L, L)
                dsti = dstbufs[b][pl.ds(g * L, L)]
                z = sdb_v[sl] + ssb_v[sl]
                ee = jnp.maximum(z, jnp.float32(NEG_SLOPE) * z)
                p = jnp.exp(ee - cshift)
                p_v[pl.ds(g * L, L)] = p
                plsc.addupdate_scatter(den_v, [dsti >> 7, dsti & 127], p)

        HALF = CHUNK // 2

        def scale_and_scatter(b):
            hb = hbufs[b]

            def row_body(r, _):
                pb = plsc.load_gather(p_v, [zeros16i + r])
                for j in range(d // L):
                    sl = pl.ds(j * L, L)
                    hb[r, sl] = hb[r, sl] * pb
                return 0

            npieces = CHUNK // L
            for q in range(npieces):
                lax.fori_loop(q * L, (q + 1) * L, row_body, 0, unroll=8)
                piece = (hb.at[pl.ds(q * L, L)],
                         acc_sh.at[dstbufs[b].at[pl.ds(q * L, L)]])
                if q < npieces - 1:
                    pltpu.async_copy(piece[0], piece[1], scsems[b],
                                     add=True)
                else:
                    pltpu.sync_copy(piece[0], piece[1], add=True)
            for q in range(npieces - 1):
                pltpu.make_async_copy(hb.at[pl.ds(q * L, L)],
                                      acc_sh.at[dstbufs[b].at[pl.ds(q * L, L)]],
                                      scsems[b]).wait()

        def do_chunk(i, b):
            nb = 1 - b
            wait_src(nb)
            wait_dst(nb)
            fire_gather(nb)
            fire_sd(nb)
            fire_ss(nb)
            wait_sd(b)
            wait_ss(b)
            score(b)
            wait_gather(b)
            fire_src(i + 2, b)
            scale_and_scatter(b)
            fire_dst(i + 2, b)

        pltpu.sync_copy(src_hbm.at[pl.ds(ebase, CHUNK)], srcslice(0))
        pltpu.sync_copy(dst_hbm.at[pl.ds(ebase, CHUNK)], dst0_v)
        fire_gather(0)
        fire_sd(0)
        fire_ss(0)
        fire_src(1, 1)
        fire_dst(1, 1)

        def pair_body(j, _):
            do_chunk(j * 2, 0)
            do_chunk(j * 2 + 1, 1)
            return 0

        lax.fori_loop(0, pairs, pair_body, 0)

        wait_sd(0)
        wait_ss(0)
        score(0)
        wait_gather(0)
        scale_and_scatter(0)
        wait_src(1)
        wait_dst(1)

        iota16 = lax.iota(jnp.int32, L)

        def fill_idx(g, _):
            dst0_v[pl.ds(g * L, L)] = iota16 + (n + g * L)
            return 0

        lax.fori_loop(0, drows // L, fill_idx, 0)
        pltpu.sync_copy(den_v, acc_sh.at[dst0_v], add=True)

        plsc.subcore_barrier()

        pltpu.sync_copy(acc_sh.at[pl.ds(rbase, rows_per_tile)],
                        out_hbm.at[c, pl.ds(rbase, rows_per_tile)])

        @pl.when(s == 0)
        def _():
            pltpu.sync_copy(acc_sh.at[pl.ds(n, drows)],
                            out_hbm.at[c, pl.ds(n, drows)])

    return sc_gat


def _fin_body(acc_ref, den_ref, o_ref):
    acc = acc_ref[0] + acc_ref[1]
    den = den_ref[0] + den_ref[1]
    o_ref[...] = jnp.maximum(acc / (den + 1e-16), 0.0)


def _finalize(acc, den, n, d, block_rows=1024):
    grid = -(-n // block_rows)
    return pl.pallas_call(
        _fin_body,
        grid=(grid,),
        in_specs=[
            pl.BlockSpec((NC, block_rows, d), lambda i: (0, i, 0)),
            pl.BlockSpec((NC, block_rows, 1), lambda i: (0, i, 0)),
        ],
        out_specs=pl.BlockSpec((block_rows, d), lambda i: (i, 0)),
        out_shape=jax.ShapeDtypeStruct((n, d), jnp.float32),
    )(acc, den)


def kernel(x, edge_src, edge_dst, W, a):
    n, d = x.shape
    e = edge_src.shape[0]
    dout = W.shape[1]
    a_dst = a[:dout]
    a_src = a[dout:]
    h, sd, ss, cmax = _project(x, W, a_dst, a_src)
    sc_gat = _make_sc_kernel(n, e, dout)
    acc = sc_gat(h, edge_src, edge_dst, sd.reshape(n), ss.reshape(n), cmax)
    den = acc[:, n:].reshape(NC, -1, 1)
    return _finalize(acc, den, n, dout)


# speedup vs baseline: 1.0203x; 1.0203x over previous
import functools

import jax
import jax.numpy as jnp
from jax import lax
from jax.experimental import pallas as pl
from jax.experimental.pallas import tpu as pltpu
from jax.experimental.pallas import tpu_sc as plsc

NEG_SLOPE = 0.2
CHUNK = 80
NC, NS, L = 2, 16, 16


def _proj_body(x_ref, w_ref, ad_ref, as_ref, h_ref, sd_ref, ss_ref, c_ref,
               msd_ref, mss_ref):
    i = pl.program_id(0)
    h = jnp.dot(x_ref[...], w_ref[...], preferred_element_type=jnp.float32)
    h_ref[...] = h
    sd = jnp.dot(h, ad_ref[...], preferred_element_type=jnp.float32)
    ss = jnp.dot(h, as_ref[...], preferred_element_type=jnp.float32)
    sd_ref[...] = sd
    ss_ref[...] = ss
    msd = jnp.broadcast_to(jnp.max(sd), (1, 128))
    mss = jnp.broadcast_to(jnp.max(ss), (1, 128))

    @pl.when(i == 0)
    def _():
        msd_ref[...] = msd
        mss_ref[...] = mss

    @pl.when(i > 0)
    def _():
        msd_ref[...] = jnp.maximum(msd_ref[...], msd)
        mss_ref[...] = jnp.maximum(mss_ref[...], mss)

    z = msd_ref[...] + mss_ref[...]
    c_ref[...] = jnp.maximum(z, jnp.float32(NEG_SLOPE) * z)


def _project(x, W, a_dst, a_src, block_rows=1000):
    n, d = x.shape
    dout = W.shape[1]
    grid = n // block_rows
    return pl.pallas_call(
        _proj_body,
        grid=(grid,),
        in_specs=[
            pl.BlockSpec((block_rows, d), lambda i: (i, 0)),
            pl.BlockSpec((d, dout), lambda i: (0, 0)),
            pl.BlockSpec((dout, 1), lambda i: (0, 0)),
            pl.BlockSpec((dout, 1), lambda i: (0, 0)),
        ],
        out_specs=[
            pl.BlockSpec((block_rows, dout), lambda i: (i, 0)),
            pl.BlockSpec((block_rows, 1), lambda i: (i, 0)),
            pl.BlockSpec((block_rows, 1), lambda i: (i, 0)),
            pl.BlockSpec((1, 128), lambda i: (0, 0)),
        ],
        out_shape=[
            jax.ShapeDtypeStruct((n, dout), jnp.float32),
            jax.ShapeDtypeStruct((n, 1), jnp.float32),
            jax.ShapeDtypeStruct((n, 1), jnp.float32),
            jax.ShapeDtypeStruct((1, 128), jnp.float32),
        ],
        scratch_shapes=[
            pltpu.VMEM((1, 128), jnp.float32),
            pltpu.VMEM((1, 128), jnp.float32),
        ],
    )(x, W, a_dst, a_src)


def _make_sc_kernel(n, e, d):
    epw = e // (NC * NS)
    nchunks = epw // CHUNK
    assert nchunks % 2 == 1, "pipeline assumes odd chunk count"
    pairs = (nchunks - 1) // 2
    groups = CHUNK // L
    drows = -(-(-(-n // 128)) // 8) * 8
    assert drows == CHUNK
    rows_per_tile = (-(-(n // NS) // 8) + 1) * 8
    assert (n // NS) * (NS - 1) // 8 * 8 + rows_per_tile >= n
    nzfull = rows_per_tile // CHUNK
    nzrem = rows_per_tile - nzfull * CHUNK

    mesh = plsc.VectorSubcoreMesh(core_axis_name="c", subcore_axis_name="s")

    @functools.partial(
        pl.kernel,
        out_type=jax.ShapeDtypeStruct((NC, n + drows, d), jnp.float32),
        mesh=mesh,
        scratch_types=[
            pltpu.VMEM((drows, 128), jnp.float32),
            pltpu.VMEM((CHUNK, d), jnp.float32),
            pltpu.VMEM((CHUNK, d), jnp.float32),
            pltpu.VMEM((2 * CHUNK,), jnp.int32),
            pltpu.VMEM((CHUNK,), jnp.int32),
            pltpu.VMEM((CHUNK,), jnp.int32),
            pltpu.VMEM((2 * CHUNK,), jnp.float32),
            pltpu.VMEM((2 * CHUNK,), jnp.float32),
            pltpu.VMEM((CHUNK,), jnp.float32),
            pltpu.VMEM((L,), jnp.float32),
            pltpu.VMEM_SHARED((n + drows, d), jnp.float32),
            pltpu.SemaphoreType.DMA,
            pltpu.SemaphoreType.DMA,
            pltpu.SemaphoreType.DMA,
            pltpu.SemaphoreType.DMA,
            pltpu.SemaphoreType.DMA,
            pltpu.SemaphoreType.DMA,
            pltpu.SemaphoreType.DMA,
            pltpu.SemaphoreType.DMA,
            pltpu.SemaphoreType.DMA,
            pltpu.SemaphoreType.DMA,
            pltpu.SemaphoreType.DMA,
            pltpu.SemaphoreType.DMA,
        ],
        compiler_params=pltpu.CompilerParams(needs_layout_passes=False),
    )
    def sc_gat(h_hbm, src_hbm, dst_hbm, sd_hbm, ss_hbm, c_hbm, out_hbm,
               den_v, h0_v, h1_v, srcb_v, dst0_v, dst1_v, sdb_v, ssb_v,
               p_v, cb_v, acc_sh,
               ssem0, ssem1, dsem0, dsem1, gsem0, gsem1,
               sdsem0, sdsem1, sssem0, sssem1, scsem0, scsem1):
        c = lax.axis_index("c")
        s = lax.axis_index("s")
        wid = s * NC + c
        ebase = wid * epw

        zeros16 = jnp.zeros((L,), jnp.float32)
        zeros16i = jnp.zeros((L,), jnp.int32)

        def zero_row(r, _):
            for j in range(d // L):
                h0_v[r, pl.ds(j * L, L)] = zeros16
            return 0

        lax.fori_loop(0, CHUNK, zero_row, 0)

        def zero_den(r, _):
            for j in range(128 // L):
                den_v[r, pl.ds(j * L, L)] = zeros16
            return 0

        lax.fori_loop(0, drows, zero_den, 0)

        rbase = (s * (n // NS)) // 8 * 8
        for k in range(nzfull):
            pltpu.sync_copy(h0_v, acc_sh.at[pl.ds(rbase + k * CHUNK, CHUNK)])
        if nzrem:
            pltpu.sync_copy(h0_v.at[pl.ds(0, nzrem)],
                            acc_sh.at[pl.ds(rbase + nzfull * CHUNK, nzrem)])

        @pl.when(s == 0)
        def _():
            pltpu.sync_copy(h0_v, acc_sh.at[pl.ds(n, drows)])

        pltpu.sync_copy(c_hbm.at[0, pl.ds(0, L)], cb_v)
        cshift = cb_v[pl.ds(0, L)]

        plsc.subcore_barrier()

        hbufs = (h0_v, h1_v)
        dstbufs = (dst0_v, dst1_v)
        ssems = (ssem0, ssem1)
        dsems = (dsem0, dsem1)
        gsems = (gsem0, gsem1)
        sdsems = (sdsem0, sdsem1)
        sssems = (sssem0, sssem1)
        scsems = (scsem0, scsem1)
        last = nchunks - 1

        def srcslice(b):
            return srcb_v.at[pl.ds(b * CHUNK, CHUNK)]

        def sdslice(b):
            return sdb_v.at[pl.ds(b * CHUNK, CHUNK)]

        def ssslice(b):
            return ssb_v.at[pl.ds(b * CHUNK, CHUNK)]

        def fire_src(ci, b):
            off = ebase + jnp.minimum(ci, last) * CHUNK
            pltpu.async_copy(src_hbm.at[pl.ds(off, CHUNK)], srcslice(b),
                             ssems[b])

        def wait_src(b):
            pltpu.make_async_copy(src_hbm.at[pl.ds(0, CHUNK)], srcslice(b),
                                  ssems[b]).wait()

        def fire_dst(ci, b):
            off = ebase + jnp.minimum(ci, last) * CHUNK
            pltpu.async_copy(dst_hbm.at[pl.ds(off, CHUNK)], dstbufs[b],
                             dsems[b])

        def wait_dst(b):
            pltpu.make_async_copy(dst_hbm.at[pl.ds(0, CHUNK)], dstbufs[b],
                                  dsems[b]).wait()

        def fire_gather(b):
            pltpu.async_copy(h_hbm.at[srcslice(b)], hbufs[b], gsems[b])

        def wait_gather(b):
            pltpu.make_async_copy(h_hbm.at[srcslice(b)], hbufs[b],
                                  gsems[b]).wait()

        def fire_sd(b):
            pltpu.async_copy(sd_hbm.at[dstbufs[b]], sdslice(b), sdsems[b])

        def wait_sd(b):
            pltpu.make_async_copy(sd_hbm.at[dstbufs[b]], sdslice(b),
                                  sdsems[b]).wait()

        def fire_ss(b):
            pltpu.async_copy(ss_hbm.at[srcslice(b)], ssslice(b), sssems[b])

        def wait_ss(b):
            pltpu.make_async_copy(ss_hbm.at[srcslice(b)], ssslice(b),
                                  sssems[b]).wait()

        def score(b):
            for g in range(groups):
                sl = pl.ds(b * CHUNK + g * L, L)
                dsti = dstbufs[b][pl.ds(g * L, L)]
                z = sdb_v[sl] + ssb_v[sl]
                ee = jnp.maximum(z, jnp.float32(NEG_SLOPE) * z)
                p = jnp.exp(ee - cshift)
                p_v[pl.ds(g * L, L)] = p
                plsc.addupdate_scatter(den_v, [dsti >> 7, dsti & 127], p)

        HALF = CHUNK // 2

        def scale_and_scatter(b):
            hb = hbufs[b]

            def row_body(r, _):
                pb = plsc.load_gather(p_v, [zeros16i + r])
                for j in range(d // L):
                    sl = pl.ds(j * L, L)
                    hb[r, sl] = hb[r, sl] * pb
                return 0

            lax.fori_loop(0, HALF, row_body, 0, unroll=16)
            pltpu.async_copy(hb.at[pl.ds(0, HALF)],
                             acc_sh.at[dstbufs[b].at[pl.ds(0, HALF)]],
                             scsems[b], add=True)
            lax.fori_loop(HALF, CHUNK, row_body, 0, unroll=16)
            pltpu.sync_copy(hb.at[pl.ds(HALF, HALF)],
                            acc_sh.at[dstbufs[b].at[pl.ds(HALF, HALF)]],
                            add=True)
            pltpu.make_async_copy(hb.at[pl.ds(0, HALF)],
                                  acc_sh.at[dstbufs[b].at[pl.ds(0, HALF)]],
                                  scsems[b]).wait()

        def do_chunk(i, b):
            nb = 1 - b
            wait_src(nb)
            wait_dst(nb)
            fire_gather(nb)
            fire_sd(nb)
            fire_ss(nb)
            wait_sd(b)
            wait_ss(b)
            score(b)
            wait_gather(b)
            fire_src(i + 2, b)
            scale_and_scatter(b)
            fire_dst(i + 2, b)

        pltpu.sync_copy(src_hbm.at[pl.ds(ebase, CHUNK)], srcslice(0))
        pltpu.sync_copy(dst_hbm.at[pl.ds(ebase, CHUNK)], dst0_v)
        fire_gather(0)
        fire_sd(0)
        fire_ss(0)
        fire_src(1, 1)
        fire_dst(1, 1)

        def pair_body(j, _):
            do_chunk(j * 2, 0)
            do_chunk(j * 2 + 1, 1)
            return 0

        lax.fori_loop(0, pairs, pair_body, 0)

        wait_sd(0)
        wait_ss(0)
        score(0)
        wait_gather(0)
        scale_and_scatter(0)
        wait_src(1)
        wait_dst(1)

        iota16 = lax.iota(jnp.int32, L)

        def fill_idx(g, _):
            dst0_v[pl.ds(g * L, L)] = iota16 + (n + g * L)
            return 0

        lax.fori_loop(0, drows // L, fill_idx, 0)
        pltpu.sync_copy(den_v, acc_sh.at[dst0_v], add=True)

        plsc.subcore_barrier()

        pltpu.sync_copy(acc_sh.at[pl.ds(rbase, rows_per_tile)],
                        out_hbm.at[c, pl.ds(rbase, rows_per_tile)])

        @pl.when(s == 0)
        def _():
            pltpu.sync_copy(acc_sh.at[pl.ds(n, drows)],
                            out_hbm.at[c, pl.ds(n, drows)])

    return sc_gat


def _fin_body(acc_ref, den_ref, o_ref):
    acc = acc_ref[0] + acc_ref[1]
    den = den_ref[0] + den_ref[1]
    o_ref[...] = jnp.maximum(acc / (den + 1e-16), 0.0)


def _finalize(acc, den, n, d, block_rows=1024):
    grid = -(-n // block_rows)
    return pl.pallas_call(
        _fin_body,
        grid=(grid,),
        in_specs=[
            pl.BlockSpec((NC, block_rows, d), lambda i: (0, i, 0)),
            pl.BlockSpec((NC, block_rows, 1), lambda i: (0, i, 0)),
        ],
        out_specs=pl.BlockSpec((block_rows, d), lambda i: (i, 0)),
        out_shape=jax.ShapeDtypeStruct((n, d), jnp.float32),
    )(acc, den)


def kernel(x, edge_src, edge_dst, W, a):
    n, d = x.shape
    e = edge_src.shape[0]
    dout = W.shape[1]
    a_dst = a[:dout]
    a_src = a[dout:]
    h, sd, ss, cmax = _project(x, W, a_dst, a_src)
    sc_gat = _make_sc_kernel(n, e, dout)
    acc = sc_gat(h, edge_src, edge_dst, sd.reshape(n), ss.reshape(n), cmax)
    den = acc[:, n:].reshape(NC, -1, 1)
    return _finalize(acc, den, n, dout)


# speedup vs baseline: 1.2442x; 1.2194x over previous
import functools

import jax
import jax.numpy as jnp
from jax import lax
from jax.experimental import pallas as pl
from jax.experimental.pallas import tpu as pltpu
from jax.experimental.pallas import tpu_sc as plsc

NEG_SLOPE = 0.2
CHUNK = 80
NC, NS, L = 2, 16, 16


def _proj_body(x_ref, w_ref, ad_ref, as_ref, h_ref, sd_ref, ss_ref, c_ref,
               msd_ref, mss_ref):
    i = pl.program_id(0)
    h = jnp.dot(x_ref[...], w_ref[...], preferred_element_type=jnp.float32)
    h_ref[...] = h
    sd = jnp.dot(h, ad_ref[...], preferred_element_type=jnp.float32)
    ss = jnp.dot(h, as_ref[...], preferred_element_type=jnp.float32)
    sd_ref[...] = sd
    ss_ref[...] = ss
    msd = jnp.broadcast_to(jnp.max(sd), (1, 128))
    mss = jnp.broadcast_to(jnp.max(ss), (1, 128))

    @pl.when(i == 0)
    def _():
        msd_ref[...] = msd
        mss_ref[...] = mss

    @pl.when(i > 0)
    def _():
        msd_ref[...] = jnp.maximum(msd_ref[...], msd)
        mss_ref[...] = jnp.maximum(mss_ref[...], mss)

    z = msd_ref[...] + mss_ref[...]
    c_ref[...] = jnp.maximum(z, jnp.float32(NEG_SLOPE) * z)


def _project(x, W, a_dst, a_src, block_rows=1000):
    n, d = x.shape
    dout = W.shape[1]
    grid = n // block_rows
    return pl.pallas_call(
        _proj_body,
        grid=(grid,),
        in_specs=[
            pl.BlockSpec((block_rows, d), lambda i: (i, 0)),
            pl.BlockSpec((d, dout), lambda i: (0, 0)),
            pl.BlockSpec((dout, 1), lambda i: (0, 0)),
            pl.BlockSpec((dout, 1), lambda i: (0, 0)),
        ],
        out_specs=[
            pl.BlockSpec((block_rows, dout), lambda i: (i, 0)),
            pl.BlockSpec((block_rows, 1), lambda i: (i, 0)),
            pl.BlockSpec((block_rows, 1), lambda i: (i, 0)),
            pl.BlockSpec((1, 128), lambda i: (0, 0)),
        ],
        out_shape=[
            jax.ShapeDtypeStruct((n, dout), jnp.float32),
            jax.ShapeDtypeStruct((n, 1), jnp.float32),
            jax.ShapeDtypeStruct((n, 1), jnp.float32),
            jax.ShapeDtypeStruct((1, 128), jnp.float32),
        ],
        scratch_shapes=[
            pltpu.VMEM((1, 128), jnp.float32),
            pltpu.VMEM((1, 128), jnp.float32),
        ],
    )(x, W, a_dst, a_src)


def _make_sc_kernel(n, e, d):
    epw = e // (NC * NS)
    nchunks = epw // CHUNK
    assert nchunks % 2 == 1, "pipeline assumes odd chunk count"
    pairs = (nchunks - 1) // 2
    groups = CHUNK // L
    drows = -(-(-(-n // 128)) // 8) * 8
    assert drows == CHUNK
    rows_per_tile = (-(-(n // NS) // 8) + 1) * 8
    assert (n // NS) * (NS - 1) // 8 * 8 + rows_per_tile >= n
    nzfull = rows_per_tile // CHUNK
    nzrem = rows_per_tile - nzfull * CHUNK

    mesh = plsc.VectorSubcoreMesh(core_axis_name="c", subcore_axis_name="s")

    @functools.partial(
        pl.kernel,
        out_type=jax.ShapeDtypeStruct((NC, n + drows, d), jnp.float32),
        mesh=mesh,
        scratch_types=[
            pltpu.VMEM((drows, 128), jnp.float32),
            pltpu.VMEM((CHUNK, d), jnp.float32),
            pltpu.VMEM((CHUNK, d), jnp.float32),
            pltpu.VMEM((2 * CHUNK,), jnp.int32),
            pltpu.VMEM((CHUNK,), jnp.int32),
            pltpu.VMEM((CHUNK,), jnp.int32),
            pltpu.VMEM((2 * CHUNK,), jnp.float32),
            pltpu.VMEM((2 * CHUNK,), jnp.float32),
            pltpu.VMEM((CHUNK,), jnp.float32),
            pltpu.VMEM((L,), jnp.float32),
            pltpu.VMEM_SHARED((n + drows, d), jnp.float32),
            pltpu.SemaphoreType.DMA,
            pltpu.SemaphoreType.DMA,
            pltpu.SemaphoreType.DMA,
            pltpu.SemaphoreType.DMA,
            pltpu.SemaphoreType.DMA,
            pltpu.SemaphoreType.DMA,
            pltpu.SemaphoreType.DMA,
            pltpu.SemaphoreType.DMA,
            pltpu.SemaphoreType.DMA,
            pltpu.SemaphoreType.DMA,
            pltpu.SemaphoreType.DMA,
            pltpu.SemaphoreType.DMA,
        ],
        compiler_params=pltpu.CompilerParams(needs_layout_passes=False),
    )
    def sc_gat(h_hbm, src_hbm, dst_hbm, sd_hbm, ss_hbm, c_hbm, out_hbm,
               den_v, h0_v, h1_v, srcb_v, dst0_v, dst1_v, sdb_v, ssb_v,
               p_v, cb_v, acc_sh,
               ssem0, ssem1, dsem0, dsem1, gsem0, gsem1,
               sdsem0, sdsem1, sssem0, sssem1, scsem0, scsem1):
        c = lax.axis_index("c")
        s = lax.axis_index("s")
        wid = s * NC + c
        ebase = wid * epw

        zeros16 = jnp.zeros((L,), jnp.float32)
        zeros16i = jnp.zeros((L,), jnp.int32)

        def zero_row(r, _):
            for j in range(d // L):
                h0_v[r, pl.ds(j * L, L)] = zeros16
            return 0

        lax.fori_loop(0, CHUNK, zero_row, 0)

        def zero_den(r, _):
            for j in range(128 // L):
                den_v[r, pl.ds(j * L, L)] = zeros16
            return 0

        lax.fori_loop(0, drows, zero_den, 0)

        rbase = (s * (n // NS)) // 8 * 8
        for k in range(nzfull):
            pltpu.sync_copy(h0_v, acc_sh.at[pl.ds(rbase + k * CHUNK, CHUNK)])
        if nzrem:
            pltpu.sync_copy(h0_v.at[pl.ds(0, nzrem)],
                            acc_sh.at[pl.ds(rbase + nzfull * CHUNK, nzrem)])

        @pl.when(s == 0)
        def _():
            pltpu.sync_copy(h0_v, acc_sh.at[pl.ds(n, drows)])

        pltpu.sync_copy(c_hbm.at[0, pl.ds(0, L)], cb_v)
        cshift = cb_v[pl.ds(0, L)]

        plsc.subcore_barrier()

        hbufs = (h0_v, h1_v)
        dstbufs = (dst0_v, dst1_v)
        ssems = (ssem0, ssem1)
        dsems = (dsem0, dsem1)
        gsems = (gsem0, gsem1)
        sdsems = (sdsem0, sdsem1)
        sssems = (sssem0, sssem1)
        scsems = (scsem0, scsem1)
        last = nchunks - 1

        def srcslice(b):
            return srcb_v.at[pl.ds(b * CHUNK, CHUNK)]

        def sdslice(b):
            return sdb_v.at[pl.ds(b * CHUNK, CHUNK)]

        def ssslice(b):
            return ssb_v.at[pl.ds(b * CHUNK, CHUNK)]

        def fire_src(ci, b):
            off = ebase + jnp.minimum(ci, last) * CHUNK
            pltpu.async_copy(src_hbm.at[pl.ds(off, CHUNK)], srcslice(b),
                             ssems[b])

        def wait_src(b):
            pltpu.make_async_copy(src_hbm.at[pl.ds(0, CHUNK)], srcslice(b),
                                  ssems[b]).wait()

        def fire_dst(ci, b):
            off = ebase + jnp.minimum(ci, last) * CHUNK
            pltpu.async_copy(dst_hbm.at[pl.ds(off, CHUNK)], dstbufs[b],
                             dsems[b])

        def wait_dst(b):
            pltpu.make_async_copy(dst_hbm.at[pl.ds(0, CHUNK)], dstbufs[b],
                                  dsems[b]).wait()

        def fire_gather(b):
            pltpu.async_copy(h_hbm.at[srcslice(b)], hbufs[b], gsems[b])

        def wait_gather(b):
            pltpu.make_async_copy(h_hbm.at[srcslice(b)], hbufs[b],
                                  gsems[b]).wait()

        def fire_sd(b):
            pltpu.async_copy(sd_hbm.at[dstbufs[b]], sdslice(b), sdsems[b])

        def wait_sd(b):
            pltpu.make_async_copy(sd_hbm.at[dstbufs[b]], sdslice(b),
                                  sdsems[b]).wait()

        def fire_ss(b):
            pltpu.async_copy(ss_hbm.at[srcslice(b)], ssslice(b), sssems[b])

        def wait_ss(b):
            pltpu.make_async_copy(ss_hbm.at[srcslice(b)], ssslice(b),
                                  sssems[b]).wait()

        def score(b):
            for g in range(groups):
                sl = pl.ds(b * CHUNK + g * L, L)
                dsti = dstbufs[b][pl.ds(g * L, L)]
                z = sdb_v[sl] + ssb_v[sl]
                ee = jnp.maximum(z, jnp.float32(NEG_SLOPE) * z)
                p = jnp.exp(ee - cshift)
                p_v[pl.ds(g * L, L)] = p
                plsc.addupdate_scatter(den_v, [dsti >> 7, dsti & 127], p)

        HALF = CHUNK // 2

        def scale_and_scatter(b):
            hb = hbufs[b]

            def row_body(r, _):
                pb = plsc.load_gather(p_v, [zeros16i + r])
                for j in range(d // L):
                    sl = pl.ds(j * L, L)
                    hb[r, sl] = hb[r, sl] * pb
                return 0

            lax.fori_loop(0, HALF, row_body, 0, unroll=8)
            pltpu.async_copy(hb.at[pl.ds(0, HALF)],
                             acc_sh.at[dstbufs[b].at[pl.ds(0, HALF)]],
                             scsems[b], add=True)
            lax.fori_loop(HALF, CHUNK, row_body, 0, unroll=8)
            pltpu.sync_copy(hb.at[pl.ds(HALF, HALF)],
                            acc_sh.at[dstbufs[b].at[pl.ds(HALF, HALF)]],
                            add=True)
            pltpu.make_async_copy(hb.at[pl.ds(0, HALF)],
                                  acc_sh.at[dstbufs[b].at[pl.ds(0, HALF)]],
                                  scsems[b]).wait()

        def do_chunk(i, b):
            nb = 1 - b
            wait_src(nb)
            wait_dst(nb)
            fire_gather(nb)
            fire_sd(nb)
            fire_ss(nb)
            wait_sd(b)
            wait_ss(b)
            score(b)
            wait_gather(b)
            fire_src(i + 2, b)
            scale_and_scatter(b)
            fire_dst(i + 2, b)

        pltpu.sync_copy(src_hbm.at[pl.ds(ebase, CHUNK)], srcslice(0))
        pltpu.sync_copy(dst_hbm.at[pl.ds(ebase, CHUNK)], dst0_v)
        fire_gather(0)
        fire_sd(0)
        fire_ss(0)
        fire_src(1, 1)
        fire_dst(1, 1)

        def pair_body(j, _):
            do_chunk(j * 2, 0)
            do_chunk(j * 2 + 1, 1)
            return 0

        lax.fori_loop(0, pairs, pair_body, 0)

        wait_sd(0)
        wait_ss(0)
        score(0)
        wait_gather(0)
        scale_and_scatter(0)
        wait_src(1)
        wait_dst(1)

        iota16 = lax.iota(jnp.int32, L)

        def fill_idx(g, _):
            dst0_v[pl.ds(g * L, L)] = iota16 + (n + g * L)
            return 0

        lax.fori_loop(0, drows // L, fill_idx, 0)
        pltpu.sync_copy(den_v, acc_sh.at[dst0_v], add=True)

        plsc.subcore_barrier()

        pltpu.sync_copy(acc_sh.at[pl.ds(rbase, rows_per_tile)],
                        out_hbm.at[c, pl.ds(rbase, rows_per_tile)])

        @pl.when(s == 0)
        def _():
            pltpu.sync_copy(acc_sh.at[pl.ds(n, drows)],
                            out_hbm.at[c, pl.ds(n, drows)])

    return sc_gat


def _fin_body(acc_ref, den_ref, o_ref):
    acc = acc_ref[0] + acc_ref[1]
    den = den_ref[0] + den_ref[1]
    o_ref[...] = jnp.maximum(acc / (den + 1e-16), 0.0)


def _finalize(acc, den, n, d, block_rows=1024):
    grid = -(-n // block_rows)
    return pl.pallas_call(
        _fin_body,
        grid=(grid,),
        in_specs=[
            pl.BlockSpec((NC, block_rows, d), lambda i: (0, i, 0)),
            pl.BlockSpec((NC, block_rows, 1), lambda i: (0, i, 0)),
        ],
        out_specs=pl.BlockSpec((block_rows, d), lambda i: (i, 0)),
        out_shape=jax.ShapeDtypeStruct((n, d), jnp.float32),
    )(acc, den)


def kernel(x, edge_src, edge_dst, W, a):
    n, d = x.shape
    e = edge_src.shape[0]
    dout = W.shape[1]
    a_dst = a[:dout]
    a_src = a[dout:]
    h, sd, ss, cmax = _project(x, W, a_dst, a_src)
    sc_gat = _make_sc_kernel(n, e, dout)
    acc = sc_gat(h, edge_src, edge_dst, sd.reshape(n), ss.reshape(n), cmax)
    den = acc[:, n:].reshape(NC, -1, 1)
    return _finalize(acc, den, n, dout)


# speedup vs baseline: 1.2673x; 1.0185x over previous
import functools

import jax
import jax.numpy as jnp
from jax import lax
from jax.experimental import pallas as pl
from jax.experimental.pallas import tpu as pltpu
from jax.experimental.pallas import tpu_sc as plsc

NEG_SLOPE = 0.2
CHUNK = 80
NC, NS, L = 2, 16, 16


def _proj_body(x_ref, w_ref, ad_ref, as_ref, h_ref, sd_ref, ss_ref, c_ref,
               msd_ref, mss_ref):
    i = pl.program_id(0)
    h = jnp.dot(x_ref[...], w_ref[...], preferred_element_type=jnp.float32)
    h_ref[...] = h
    sd = jnp.dot(h, ad_ref[...], preferred_element_type=jnp.float32)
    ss = jnp.dot(h, as_ref[...], preferred_element_type=jnp.float32)
    sd_ref[...] = sd
    ss_ref[...] = ss
    msd = jnp.broadcast_to(jnp.max(sd), (1, 128))
    mss = jnp.broadcast_to(jnp.max(ss), (1, 128))

    @pl.when(i == 0)
    def _():
        msd_ref[...] = msd
        mss_ref[...] = mss

    @pl.when(i > 0)
    def _():
        msd_ref[...] = jnp.maximum(msd_ref[...], msd)
        mss_ref[...] = jnp.maximum(mss_ref[...], mss)

    z = msd_ref[...] + mss_ref[...]
    c_ref[...] = jnp.maximum(z, jnp.float32(NEG_SLOPE) * z)


def _project(x, W, a_dst, a_src, block_rows=1000):
    n, d = x.shape
    dout = W.shape[1]
    grid = n // block_rows
    return pl.pallas_call(
        _proj_body,
        grid=(grid,),
        in_specs=[
            pl.BlockSpec((block_rows, d), lambda i: (i, 0)),
            pl.BlockSpec((d, dout), lambda i: (0, 0)),
            pl.BlockSpec((dout, 1), lambda i: (0, 0)),
            pl.BlockSpec((dout, 1), lambda i: (0, 0)),
        ],
        out_specs=[
            pl.BlockSpec((block_rows, dout), lambda i: (i, 0)),
            pl.BlockSpec((block_rows, 1), lambda i: (i, 0)),
            pl.BlockSpec((block_rows, 1), lambda i: (i, 0)),
            pl.BlockSpec((1, 128), lambda i: (0, 0)),
        ],
        out_shape=[
            jax.ShapeDtypeStruct((n, dout), jnp.float32),
            jax.ShapeDtypeStruct((n, 1), jnp.float32),
            jax.ShapeDtypeStruct((n, 1), jnp.float32),
            jax.ShapeDtypeStruct((1, 128), jnp.float32),
        ],
        scratch_shapes=[
            pltpu.VMEM((1, 128), jnp.float32),
            pltpu.VMEM((1, 128), jnp.float32),
        ],
    )(x, W, a_dst, a_src)


def _make_sc_kernel(n, e, d):
    epw = e // (NC * NS)
    nchunks = epw // CHUNK
    assert nchunks % 2 == 1, "pipeline assumes odd chunk count"
    pairs = (nchunks - 1) // 2
    groups = CHUNK // L
    drows = -(-(-(-n // 128)) // 8) * 8
    assert drows == CHUNK
    rows_per_tile = (-(-(n // NS) // 8) + 1) * 8
    assert (n // NS) * (NS - 1) // 8 * 8 + rows_per_tile >= n
    nzfull = rows_per_tile // CHUNK
    nzrem = rows_per_tile - nzfull * CHUNK

    mesh = plsc.VectorSubcoreMesh(core_axis_name="c", subcore_axis_name="s")

    @functools.partial(
        pl.kernel,
        out_type=jax.ShapeDtypeStruct((NC, n + drows, d), jnp.float32),
        mesh=mesh,
        scratch_types=[
            pltpu.VMEM((drows, 128), jnp.float32),
            pltpu.VMEM((CHUNK, d), jnp.float32),
            pltpu.VMEM((CHUNK, d), jnp.float32),
            pltpu.VMEM((2 * CHUNK,), jnp.int32),
            pltpu.VMEM((CHUNK,), jnp.int32),
            pltpu.VMEM((CHUNK,), jnp.int32),
            pltpu.VMEM((2 * CHUNK,), jnp.float32),
            pltpu.VMEM((2 * CHUNK,), jnp.float32),
            pltpu.VMEM((CHUNK,), jnp.float32),
            pltpu.VMEM((L,), jnp.float32),
            pltpu.VMEM_SHARED((n + drows, d), jnp.float32),
            pltpu.SemaphoreType.DMA,
            pltpu.SemaphoreType.DMA,
            pltpu.SemaphoreType.DMA,
            pltpu.SemaphoreType.DMA,
            pltpu.SemaphoreType.DMA,
            pltpu.SemaphoreType.DMA,
            pltpu.SemaphoreType.DMA,
            pltpu.SemaphoreType.DMA,
            pltpu.SemaphoreType.DMA,
            pltpu.SemaphoreType.DMA,
            pltpu.SemaphoreType.DMA,
            pltpu.SemaphoreType.DMA,
        ],
        compiler_params=pltpu.CompilerParams(needs_layout_passes=False),
    )
    def sc_gat(h_hbm, src_hbm, dst_hbm, sd_hbm, ss_hbm, c_hbm, out_hbm,
               den_v, h0_v, h1_v, srcb_v, dst0_v, dst1_v, sdb_v, ssb_v,
               p_v, cb_v, acc_sh,
               ssem0, ssem1, dsem0, dsem1, gsem0, gsem1,
               sdsem0, sdsem1, sssem0, sssem1, scsem0, scsem1):
        c = lax.axis_index("c")
        s = lax.axis_index("s")
        wid = s * NC + c
        ebase = wid * epw

        zeros16 = jnp.zeros((L,), jnp.float32)
        zeros16i = jnp.zeros((L,), jnp.int32)

        def zero_row(r, _):
            for j in range(d // L):
                h0_v[r, pl.ds(j * L, L)] = zeros16
            return 0

        lax.fori_loop(0, CHUNK, zero_row, 0)

        def zero_den(r, _):
            for j in range(128 // L):
                den_v[r, pl.ds(j * L, L)] = zeros16
            return 0

        lax.fori_loop(0, drows, zero_den, 0)

        rbase = (s * (n // NS)) // 8 * 8
        for k in range(nzfull):
            pltpu.sync_copy(h0_v, acc_sh.at[pl.ds(rbase + k * CHUNK, CHUNK)])
        if nzrem:
            pltpu.sync_copy(h0_v.at[pl.ds(0, nzrem)],
                            acc_sh.at[pl.ds(rbase + nzfull * CHUNK, nzrem)])

        @pl.when(s == 0)
        def _():
            pltpu.sync_copy(h0_v, acc_sh.at[pl.ds(n, drows)])

        pltpu.sync_copy(c_hbm.at[0, pl.ds(0, L)], cb_v)
        cshift = cb_v[pl.ds(0, L)]

        plsc.subcore_barrier()

        hbufs = (h0_v, h1_v)
        dstbufs = (dst0_v, dst1_v)
        ssems = (ssem0, ssem1)
        dsems = (dsem0, dsem1)
        gsems = (gsem0, gsem1)
        sdsems = (sdsem0, sdsem1)
        sssems = (sssem0, sssem1)
        scsems = (scsem0, scsem1)
        last = nchunks - 1

        def srcslice(b):
            return srcb_v.at[pl.ds(b * CHUNK, CHUNK)]

        def sdslice(b):
            return sdb_v.at[pl.ds(b * CHUNK, CHUNK)]

        def ssslice(b):
            return ssb_v.at[pl.ds(b * CHUNK, CHUNK)]

        def fire_src(ci, b):
            off = ebase + jnp.minimum(ci, last) * CHUNK
            pltpu.async_copy(src_hbm.at[pl.ds(off, CHUNK)], srcslice(b),
                             ssems[b])

        def wait_src(b):
            pltpu.make_async_copy(src_hbm.at[pl.ds(0, CHUNK)], srcslice(b),
                                  ssems[b]).wait()

        def fire_dst(ci, b):
            off = ebase + jnp.minimum(ci, last) * CHUNK
            pltpu.async_copy(dst_hbm.at[pl.ds(off, CHUNK)], dstbufs[b],
                             dsems[b])

        def wait_dst(b):
            pltpu.make_async_copy(dst_hbm.at[pl.ds(0, CHUNK)], dstbufs[b],
                                  dsems[b]).wait()

        def fire_gather(b):
            pltpu.async_copy(h_hbm.at[srcslice(b)], hbufs[b], gsems[b])

        def wait_gather(b):
            pltpu.make_async_copy(h_hbm.at[srcslice(b)], hbufs[b],
                                  gsems[b]).wait()

        def fire_sd(b):
            pltpu.async_copy(sd_hbm.at[dstbufs[b]], sdslice(b), sdsems[b])

        def wait_sd(b):
            pltpu.make_async_copy(sd_hbm.at[dstbufs[b]], sdslice(b),
                                  sdsems[b]).wait()

        def fire_ss(b):
            pltpu.async_copy(ss_hbm.at[srcslice(b)], ssslice(b), sssems[b])

        def wait_ss(b):
            pltpu.make_async_copy(ss_hbm.at[srcslice(b)], ssslice(b),
                                  sssems[b]).wait()

        def score(b):
            for g in range(groups):
                sl = pl.ds(b * CHUNK + g * L, L)
                dsti = dstbufs[b][pl.ds(g * L, L)]
                z = sdb_v[sl] + ssb_v[sl]
                ee = jnp.maximum(z, jnp.float32(NEG_SLOPE) * z)
                p = jnp.exp(ee - cshift)
                p_v[pl.ds(g * L, L)] = p
                plsc.addupdate_scatter(den_v, [dsti >> 7, dsti & 127], p)

        SPLIT = 48

        def scale_and_scatter(b):
            hb = hbufs[b]

            def row_body(r, _):
                pb = plsc.load_gather(p_v, [zeros16i + r])
                for j in range(d // L):
                    sl = pl.ds(j * L, L)
                    hb[r, sl] = hb[r, sl] * pb
                return 0

            lax.fori_loop(0, SPLIT, row_body, 0, unroll=8)
            pltpu.async_copy(hb.at[pl.ds(0, SPLIT)],
                             acc_sh.at[dstbufs[b].at[pl.ds(0, SPLIT)]],
                             scsems[b], add=True)
            lax.fori_loop(SPLIT, CHUNK, row_body, 0, unroll=8)
            pltpu.sync_copy(hb.at[pl.ds(SPLIT, CHUNK - SPLIT)],
                            acc_sh.at[dstbufs[b].at[pl.ds(SPLIT, CHUNK - SPLIT)]],
                            add=True)
            pltpu.make_async_copy(hb.at[pl.ds(0, SPLIT)],
                                  acc_sh.at[dstbufs[b].at[pl.ds(0, SPLIT)]],
                                  scsems[b]).wait()

        def do_chunk(i, b):
            nb = 1 - b
            wait_src(nb)
            wait_dst(nb)
            fire_gather(nb)
            fire_sd(nb)
            fire_ss(nb)
            wait_sd(b)
            wait_ss(b)
            score(b)
            wait_gather(b)
            fire_src(i + 2, b)
            scale_and_scatter(b)
            fire_dst(i + 2, b)

        pltpu.sync_copy(src_hbm.at[pl.ds(ebase, CHUNK)], srcslice(0))
        pltpu.sync_copy(dst_hbm.at[pl.ds(ebase, CHUNK)], dst0_v)
        fire_gather(0)
        fire_sd(0)
        fire_ss(0)
        fire_src(1, 1)
        fire_dst(1, 1)

        def pair_body(j, _):
            do_chunk(j * 2, 0)
            do_chunk(j * 2 + 1, 1)
            return 0

        lax.fori_loop(0, pairs, pair_body, 0)

        wait_sd(0)
        wait_ss(0)
        score(0)
        wait_gather(0)
        scale_and_scatter(0)
        wait_src(1)
        wait_dst(1)

        iota16 = lax.iota(jnp.int32, L)

        def fill_idx(g, _):
            dst0_v[pl.ds(g * L, L)] = iota16 + (n + g * L)
            return 0

        lax.fori_loop(0, drows // L, fill_idx, 0)
        pltpu.sync_copy(den_v, acc_sh.at[dst0_v], add=True)

        plsc.subcore_barrier()

        pltpu.sync_copy(acc_sh.at[pl.ds(rbase, rows_per_tile)],
                        out_hbm.at[c, pl.ds(rbase, rows_per_tile)])

        @pl.when(s == 0)
        def _():
            pltpu.sync_copy(acc_sh.at[pl.ds(n, drows)],
                            out_hbm.at[c, pl.ds(n, drows)])

    return sc_gat


def _fin_body(acc_ref, den_ref, o_ref):
    acc = acc_ref[0] + acc_ref[1]
    den = den_ref[0] + den_ref[1]
    o_ref[...] = jnp.maximum(acc / (den + 1e-16), 0.0)


def _finalize(acc, den, n, d, block_rows=1024):
    grid = -(-n // block_rows)
    return pl.pallas_call(
        _fin_body,
        grid=(grid,),
        in_specs=[
            pl.BlockSpec((NC, block_rows, d), lambda i: (0, i, 0)),
            pl.BlockSpec((NC, block_rows, 1), lambda i: (0, i, 0)),
        ],
        out_specs=pl.BlockSpec((block_rows, d), lambda i: (i, 0)),
        out_shape=jax.ShapeDtypeStruct((n, d), jnp.float32),
    )(acc, den)


def kernel(x, edge_src, edge_dst, W, a):
    n, d = x.shape
    e = edge_src.shape[0]
    dout = W.shape[1]
    a_dst = a[:dout]
    a_src = a[dout:]
    h, sd, ss, cmax = _project(x, W, a_dst, a_src)
    sc_gat = _make_sc_kernel(n, e, dout)
    acc = sc_gat(h, edge_src, edge_dst, sd.reshape(n), ss.reshape(n), cmax)
    den = acc[:, n:].reshape(NC, -1, 1)
    return _finalize(acc, den, n, dout)
